# baseline (speedup 1.0000x reference)
import functools

import jax
import jax.numpy as jnp
from jax import lax
from jax.experimental import pallas as pl
from jax.experimental.pallas import tpu as pltpu
from jax.experimental.pallas import tpu_sc as plsc

N = 262144
F = 16
C = 2 * F
Q = 64
Q3 = Q * Q * Q
NPC = N * C


_BN = 2048


def _prep_body(pts_ref, freq_ref, base_ref, fx_ref, fy_ref, fz_ref):
    s = pl.program_id(0)
    f = pl.program_id(1)
    p = pts_ref[...]
    fsel = lax.broadcasted_iota(jnp.int32, (1, F), 1) == f
    fval = jnp.sum(jnp.where(fsel, freq_ref[...], 0.0))
    proj = p * fval

    def emit(co):
        g = co * (0.5 * (Q - 1)) + (0.5 * (Q - 1))
        g0 = jnp.clip(jnp.floor(g), 0.0, Q - 2)
        i0 = g0.astype(jnp.int32)
        frc = g - g0
        base = (i0[:, 2] * Q + i0[:, 1]) * Q + i0[:, 0]
        base_ref[0, 0, :] = base
        fx_ref[0, 0, :] = frc[:, 0]
        fy_ref[0, 0, :] = frc[:, 1]
        fz_ref[0, 0, :] = frc[:, 2]

    @pl.when(s == 0)
    def _():
        emit(jnp.sin(proj))

    @pl.when(s == 1)
    def _():
        emit(jnp.cos(proj))


def _prep(points, freqs):
    grid = (2, F, N // _BN)
    out_shapes = [
        jax.ShapeDtypeStruct((C, 1, N), jnp.int32),
        jax.ShapeDtypeStruct((C, 1, N), jnp.float32),
        jax.ShapeDtypeStruct((C, 1, N), jnp.float32),
        jax.ShapeDtypeStruct((C, 1, N), jnp.float32),
    ]
    in_specs = [
        pl.BlockSpec((_BN, 3), lambda s, f, i: (i, 0)),
        pl.BlockSpec((1, F), lambda s, f, i: (0, 0)),
    ]
    out_specs = [
        pl.BlockSpec((1, 1, _BN), lambda s, f, i: (s * F + f, 0, i))
    ] * 4
    return pl.pallas_call(
        _prep_body,
        grid=grid,
        in_specs=in_specs,
        out_specs=out_specs,
        out_shape=out_shapes,
    )(points, freqs.reshape(1, F))


_NW = 32
_PW = N // _NW
_NSUB = _PW // 128
_STG = Q3 // 16
_OFFS = (0, 64, 4096, 4160)
_VL = Q3 - 4160


def _sc_body(cv_hbm, base_hbm, fx_hbm, fy_hbm, fz_hbm, out_hbm,
             idx_v, idx1_v, fx_v, fy_v, fz_v, o_v, c_v, shared, sem):
    cid = lax.axis_index("c")
    sid = lax.axis_index("s")
    wid = sid * 2 + cid
    lane = lax.iota(jnp.int32, 16)
    views = [shared.at[pl.ds(off, _VL)] for off in _OFFS]

    def channel(ch, carry):
        plsc.subcore_barrier()
        src0 = pl.multiple_of(ch * Q3 + sid * _STG, _STG)
        pltpu.sync_copy(cv_hbm.at[pl.ds(src0, _STG)], shared.at[pl.ds(sid * _STG, _STG)])
        plsc.subcore_barrier()

        off = pl.multiple_of(ch * N + wid * _PW, _PW)
        row0 = pl.multiple_of(off // 128, _NSUB)
        pltpu.sync_copy(base_hbm.at[pl.ds(row0, _NSUB)], idx_v)
        pltpu.sync_copy(fx_hbm.at[pl.ds(off, _PW)], fx_v)
        pltpu.sync_copy(fy_hbm.at[pl.ds(off, _PW)], fy_v)
        pltpu.sync_copy(fz_hbm.at[pl.ds(off, _PW)], fz_v)

        def bump(i, carry2):
            i16 = i * 16
            idx1_v[0, pl.ds(i16, 16)] = idx_v[pl.ds(i16 // 128, 1), pl.ds(i16 % 128, 16)].reshape(16) + 1
            return carry2

        lax.fori_loop(0, _PW // 16, bump, 0)

        def sub(j, carry2):
            cps = []
            for v in range(4):
                cps.append(pltpu.async_copy(
                    views[v].at[idx_v.at[j]],
                    c_v.at[pl.ds((2 * v) * 128, 128)], sem))
                cps.append(pltpu.async_copy(
                    views[v].at[idx1_v.at[0, pl.ds(j * 128, 128)]],
                    c_v.at[pl.ds((2 * v + 1) * 128, 128)], sem))
            for cp in cps:
                cp.wait()

            def group(g, carry3):
                i16 = j * 128 + g * 16
                fx = fx_v[pl.ds(i16, 16)]
                fy = fy_v[pl.ds(i16, 16)]
                fz = fz_v[pl.ds(i16, 16)]
                g16 = g * 16
                x00 = c_v[pl.ds(g16, 16)]
                x00 = x00 + fx * (c_v[pl.ds(128 + g16, 16)] - x00)
                x01 = c_v[pl.ds(256 + g16, 16)]
                x01 = x01 + fx * (c_v[pl.ds(384 + g16, 16)] - x01)
                x10 = c_v[pl.ds(512 + g16, 16)]
                x10 = x10 + fx * (c_v[pl.ds(640 + g16, 16)] - x10)
                x11 = c_v[pl.ds(768 + g16, 16)]
                x11 = x11 + fx * (c_v[pl.ds(896 + g16, 16)] - x11)
                y0 = x00 + fy * (x01 - x00)
                y1 = x10 + fy * (x11 - x10)
                o_v[pl.ds(i16, 16)] = y0 + fz * (y1 - y0)
                return carry3

            lax.fori_loop(0, 8, group, 0, unroll=True)
            return carry2

        lax.fori_loop(0, _NSUB, sub, 0)
        pltpu.sync_copy(o_v, out_hbm.at[pl.ds(off, _PW)])
        return carry

    lax.fori_loop(0, C, channel, 0)


@functools.cache
def _sc_gather():
    return pl.kernel(
        _sc_body,
        mesh=plsc.VectorSubcoreMesh(core_axis_name="c", subcore_axis_name="s"),
        out_type=jax.ShapeDtypeStruct((NPC,), jnp.float32),
        scratch_types=[
            pltpu.VMEM((_NSUB, 128), jnp.int32),
            pltpu.VMEM((1, _PW), jnp.int32),
            pltpu.VMEM((_PW,), jnp.float32),
            pltpu.VMEM((_PW,), jnp.float32),
            pltpu.VMEM((_PW,), jnp.float32),
            pltpu.VMEM((_PW,), jnp.float32),
            pltpu.VMEM((1024,), jnp.float32),
            pltpu.VMEM_SHARED((Q3,), jnp.float32),
            pltpu.SemaphoreType.DMA,
        ],
    )


def kernel(points, freqs, cv):
    base, fx, fy, fz = _prep(points, freqs)
    feats = _sc_gather()(
        cv.reshape(C * Q3),
        base.reshape(NPC // 128, 128),
        fx.reshape(NPC), fy.reshape(NPC), fz.reshape(NPC),
    )
    feats_t = feats.reshape(C, N).T
    return jnp.concatenate([points, feats_t], axis=1)


# speedup vs baseline: 10.7635x; 10.7635x over previous
import functools

import jax
import jax.numpy as jnp
from jax import lax
from jax.experimental import pallas as pl
from jax.experimental.pallas import tpu as pltpu
from jax.experimental.pallas import tpu_sc as plsc

N = 262144
F = 16
C = 2 * F
Q = 64
Q3 = Q * Q * Q
NPC = N * C


_SUB = 16
_NS16 = N // _SUB
_BN = 1024


def _prep_body(pts_ref, freq_ref, base_ref, fx_ref, fy_ref, fz_ref):
    s = pl.program_id(0)
    f = pl.program_id(1)
    p = pts_ref[...]
    fsel = lax.broadcasted_iota(jnp.int32, (1, F), 1) == f
    fval = jnp.sum(jnp.where(fsel, freq_ref[...], 0.0))
    proj = p * fval

    def emit(co):
        g = co * (0.5 * (Q - 1)) + (0.5 * (Q - 1))
        g0 = jnp.clip(jnp.floor(g), 0.0, Q - 2)
        i0 = g0.astype(jnp.int32)
        frc = g - g0
        base = (i0[32:48, :] * Q + i0[16:32, :]) * Q + i0[0:16, :]
        base_ref[0, :, :] = base
        fx_ref[0, :, :] = frc[0:16, :]
        fy_ref[0, :, :] = frc[16:32, :]
        fz_ref[0, :, :] = frc[32:48, :]

    @pl.when(s == 0)
    def _():
        emit(jnp.sin(proj))

    @pl.when(s == 1)
    def _():
        emit(jnp.cos(proj))


def _prep(points, freqs):
    grid = (2, F, _NS16 // _BN)
    out_shapes = [
        jax.ShapeDtypeStruct((C, _SUB, _NS16), jnp.int32),
        jax.ShapeDtypeStruct((C, _SUB, _NS16), jnp.float32),
        jax.ShapeDtypeStruct((C, _SUB, _NS16), jnp.float32),
        jax.ShapeDtypeStruct((C, _SUB, _NS16), jnp.float32),
    ]
    in_specs = [
        pl.BlockSpec((3 * _SUB, _BN), lambda s, f, i: (0, i)),
        pl.BlockSpec((1, F), lambda s, f, i: (0, 0)),
    ]
    out_specs = [
        pl.BlockSpec((1, _SUB, _BN), lambda s, f, i: (s * F + f, 0, i))
    ] * 4
    pts48 = points.T.reshape(3 * _SUB, _NS16)
    return pl.pallas_call(
        _prep_body,
        grid=grid,
        in_specs=in_specs,
        out_specs=out_specs,
        out_shape=out_shapes,
    )(pts48, freqs.reshape(1, F))


_NW = 32
_PW = N // _NW
_NSUB = _PW // 128
_STG = Q3 // 16
_OFFS = (0, 64, 4096, 4160)
_VL = Q3 - 4160


def _sc_body(cv_hbm, base_hbm, fx_hbm, fy_hbm, fz_hbm, out_hbm,
             idx_v, idx1_v, fx_v, fy_v, fz_v, o_v, c_v, shared, sem):
    cid = lax.axis_index("c")
    sid = lax.axis_index("s")
    wid = sid * 2 + cid
    lane = lax.iota(jnp.int32, 16)
    views = [shared.at[pl.ds(off, _VL)] for off in _OFFS]

    def channel(ch, carry):
        plsc.subcore_barrier()
        src0 = pl.multiple_of(ch * Q3 + sid * _STG, _STG)
        pltpu.sync_copy(cv_hbm.at[pl.ds(src0, _STG)], shared.at[pl.ds(sid * _STG, _STG)])
        plsc.subcore_barrier()

        off = pl.multiple_of(ch * N + wid * _PW, _PW)
        row0 = pl.multiple_of(off // 128, _NSUB)
        pltpu.sync_copy(base_hbm.at[pl.ds(row0, _NSUB)], idx_v)
        pltpu.sync_copy(fx_hbm.at[pl.ds(off, _PW)], fx_v)
        pltpu.sync_copy(fy_hbm.at[pl.ds(off, _PW)], fy_v)
        pltpu.sync_copy(fz_hbm.at[pl.ds(off, _PW)], fz_v)

        def bump(i, carry2):
            i16 = i * 16
            idx1_v[0, pl.ds(i16, 16)] = idx_v[pl.ds(i16 // 128, 1), pl.ds(i16 % 128, 16)].reshape(16) + 1
            return carry2

        lax.fori_loop(0, _PW // 16, bump, 0)

        def sub(j, carry2):
            cps = []
            for v in range(4):
                cps.append(pltpu.async_copy(
                    views[v].at[idx_v.at[j]],
                    c_v.at[pl.ds((2 * v) * 128, 128)], sem))
                cps.append(pltpu.async_copy(
                    views[v].at[idx1_v.at[0, pl.ds(j * 128, 128)]],
                    c_v.at[pl.ds((2 * v + 1) * 128, 128)], sem))
            for cp in cps:
                cp.wait()

            def group(g, carry3):
                i16 = j * 128 + g * 16
                fx = fx_v[pl.ds(i16, 16)]
                fy = fy_v[pl.ds(i16, 16)]
                fz = fz_v[pl.ds(i16, 16)]
                g16 = g * 16
                x00 = c_v[pl.ds(g16, 16)]
                x00 = x00 + fx * (c_v[pl.ds(128 + g16, 16)] - x00)
                x01 = c_v[pl.ds(256 + g16, 16)]
                x01 = x01 + fx * (c_v[pl.ds(384 + g16, 16)] - x01)
                x10 = c_v[pl.ds(512 + g16, 16)]
                x10 = x10 + fx * (c_v[pl.ds(640 + g16, 16)] - x10)
                x11 = c_v[pl.ds(768 + g16, 16)]
                x11 = x11 + fx * (c_v[pl.ds(896 + g16, 16)] - x11)
                y0 = x00 + fy * (x01 - x00)
                y1 = x10 + fy * (x11 - x10)
                o_v[pl.ds(i16, 16)] = y0 + fz * (y1 - y0)
                return carry3

            lax.fori_loop(0, 8, group, 0, unroll=True)
            return carry2

        lax.fori_loop(0, _NSUB, sub, 0)
        pltpu.sync_copy(o_v, out_hbm.at[pl.ds(off, _PW)])
        return carry

    lax.fori_loop(0, C, channel, 0)


@functools.cache
def _sc_gather():
    return pl.kernel(
        _sc_body,
        mesh=plsc.VectorSubcoreMesh(core_axis_name="c", subcore_axis_name="s"),
        out_type=jax.ShapeDtypeStruct((NPC,), jnp.float32),
        scratch_types=[
            pltpu.VMEM((_NSUB, 128), jnp.int32),
            pltpu.VMEM((1, _PW), jnp.int32),
            pltpu.VMEM((_PW,), jnp.float32),
            pltpu.VMEM((_PW,), jnp.float32),
            pltpu.VMEM((_PW,), jnp.float32),
            pltpu.VMEM((_PW,), jnp.float32),
            pltpu.VMEM((1024,), jnp.float32),
            pltpu.VMEM_SHARED((Q3,), jnp.float32),
            pltpu.SemaphoreType.DMA,
        ],
    )


def kernel(points, freqs, cv):
    base, fx, fy, fz = _prep(points, freqs)
    feats = _sc_gather()(
        cv.reshape(C * Q3),
        base.reshape(NPC // 128, 128),
        fx.reshape(NPC), fy.reshape(NPC), fz.reshape(NPC),
    )
    feats_t = feats.reshape(C, N).T
    return jnp.concatenate([points, feats_t], axis=1)
